# baseline (device time: 15083 ns/iter reference)
import jax
import jax.numpy as jnp
from jax import lax
from jax.experimental import pallas as pl
from jax.experimental.pallas import tpu as pltpu

C = 4


def kernel(partial, resid, gamma):
    m, d = resid.shape
    qm = m // 4
    cm = qm // C
    gamma2d = gamma.reshape(1, d)

    def body(p_ref, r_ref, g_ref, out_ref,
             comm_ref, acc_ref, psend_ref, pmine_ref, rq_ref, outq_ref,
             in_sems, out_sems, send_sems, recv_sems, ready_sems):
        my_x = lax.axis_index("x")
        my_y = lax.axis_index("y")
        my_z = lax.axis_index("z")
        q = 2 * (my_x ^ my_z) + (my_y ^ my_z)
        row0 = q * qm
        partner_z = (my_x, my_y, 1 - my_z)
        partner_x = (1 - my_x, my_y, my_z)
        partner_y = (my_x, 1 - my_y, my_z)

        cp_send = []
        for c in range(C):
            cp = pltpu.make_async_copy(
                p_ref.at[0, pl.ds((3 - q) * qm + c * cm, cm), :],
                psend_ref.at[pl.ds(c * cm, cm), :],
                in_sems.at[3 + c])
            cp.start()
            cp_send.append(cp)
        cp_mine = pltpu.make_async_copy(
            p_ref.at[0, pl.ds(row0, qm), :], pmine_ref, in_sems.at[1])
        cp_r = pltpu.make_async_copy(
            r_ref.at[pl.ds(row0, qm), :], rq_ref, in_sems.at[2])
        cp_mine.start()
        cp_r.start()

        barrier_sem = pltpu.get_barrier_semaphore()
        pl.semaphore_signal(
            barrier_sem, inc=1, device_id=partner_z,
            device_id_type=pl.DeviceIdType.MESH,
        )
        pl.semaphore_signal(
            ready_sems.at[0], inc=1, device_id=partner_x,
            device_id_type=pl.DeviceIdType.MESH,
        )
        pl.semaphore_signal(
            ready_sems.at[1], inc=1, device_id=partner_y,
            device_id_type=pl.DeviceIdType.MESH,
        )
        pl.semaphore_wait(barrier_sem, 1)

        ph1 = []
        for c in range(C):
            cp_send[c].wait()
            r = pltpu.make_async_remote_copy(
                src_ref=psend_ref.at[pl.ds(c * cm, cm), :],
                dst_ref=comm_ref.at[pl.ds(c * cm, cm), :],
                send_sem=send_sems.at[0, c],
                recv_sem=recv_sems.at[0, c],
                device_id=partner_z,
                device_id_type=pl.DeviceIdType.MESH,
            )
            r.start()
            ph1.append(r)

        cp_mine.wait()
        cp_r.wait()
        acc_ref[...] = pmine_ref[...] + rq_ref[...]

        ph2 = []
        out_cps = []
        for c in range(C):
            ph1[c].wait_recv()
            if c == 0:
                pl.semaphore_wait(ready_sems.at[0], 1)
                pl.semaphore_wait(ready_sems.at[1], 1)
            y = acc_ref[pl.ds(c * cm, cm), :] + comm_ref[pl.ds(c * cm, cm), :]
            inv = lax.rsqrt(jnp.mean(y * y, axis=-1, keepdims=True) + 1e-6)
            outq_ref[pl.ds(c * cm, cm), :] = y * inv * g_ref[...]

            src = outq_ref.at[pl.ds(c * cm, cm), :]
            dst = out_ref.at[pl.ds(row0 + c * cm, cm), :]
            cp = pltpu.make_async_copy(src, dst, out_sems.at[c])
            cp.start()
            out_cps.append(cp)
            for si, nbr in ((1, partner_z), (2, partner_x), (3, partner_y)):
                r = pltpu.make_async_remote_copy(
                    src_ref=src,
                    dst_ref=dst,
                    send_sem=send_sems.at[si, c],
                    recv_sem=recv_sems.at[si, c],
                    device_id=nbr,
                    device_id_type=pl.DeviceIdType.MESH,
                )
                r.start()
                ph2.append(r)

        for si, qq in ((1, 3 - q), (2, q ^ 2), (3, q ^ 1)):
            for c in range(C):
                dst = out_ref.at[pl.ds(qq * qm + c * cm, cm), :]
                rr = pltpu.make_async_remote_copy(
                    src_ref=dst,
                    dst_ref=dst,
                    send_sem=send_sems.at[si, c],
                    recv_sem=recv_sems.at[si, c],
                    device_id=partner_z,
                    device_id_type=pl.DeviceIdType.MESH,
                )
                rr.wait_recv()
        for cp in out_cps:
            cp.wait()
        for r in ph1:
            r.wait_send()
        for r in ph2:
            r.wait_send()

    return pl.pallas_call(
        body,
        out_shape=jax.ShapeDtypeStruct((m, d), jnp.float32),
        in_specs=[
            pl.BlockSpec(memory_space=pl.ANY),
            pl.BlockSpec(memory_space=pl.ANY),
            pl.BlockSpec(memory_space=pltpu.VMEM),
        ],
        out_specs=pl.BlockSpec(memory_space=pl.ANY),
        scratch_shapes=[
            pltpu.VMEM((qm, d), jnp.float32),
            pltpu.VMEM((qm, d), jnp.float32),
            pltpu.VMEM((qm, d), jnp.float32),
            pltpu.VMEM((qm, d), jnp.float32),
            pltpu.VMEM((qm, d), jnp.float32),
            pltpu.VMEM((qm, d), jnp.float32),
            pltpu.SemaphoreType.DMA((3 + C,)),
            pltpu.SemaphoreType.DMA((C,)),
            pltpu.SemaphoreType.DMA((4, C)),
            pltpu.SemaphoreType.DMA((4, C)),
            pltpu.SemaphoreType.REGULAR((2,)),
        ],
        compiler_params=pltpu.CompilerParams(collective_id=0),
    )(partial, resid, gamma2d)


# device time: 14772 ns/iter; 1.0211x vs baseline; 1.0211x over previous
import jax
import jax.numpy as jnp
from jax import lax
from jax.experimental import pallas as pl
from jax.experimental.pallas import tpu as pltpu

C = 4


def kernel(partial, resid, gamma):
    m, d = resid.shape
    qm = m // 4
    cm = qm // C
    gamma2d = gamma.reshape(1, d)

    def body(p_ref, r_ref, g_ref, out_ref,
             comm_ref, acc_ref, psend_ref, pmine_ref, rq_ref, outq_ref,
             in_sems, out_sems, send_sems, recv_sems, ready_sems):
        my_x = lax.axis_index("x")
        my_y = lax.axis_index("y")
        my_z = lax.axis_index("z")
        q = 2 * (my_x ^ my_z) + (my_y ^ my_z)
        row0 = q * qm
        partner_z = (my_x, my_y, 1 - my_z)
        partner_x = (1 - my_x, my_y, my_z)
        partner_y = (my_x, 1 - my_y, my_z)

        cp_send = pltpu.make_async_copy(
            p_ref.at[0, pl.ds((3 - q) * qm, qm), :], psend_ref, in_sems.at[0])
        cp_send.start()
        cp_mine = pltpu.make_async_copy(
            p_ref.at[0, pl.ds(row0, qm), :], pmine_ref, in_sems.at[1])
        cp_r = pltpu.make_async_copy(
            r_ref.at[pl.ds(row0, qm), :], rq_ref, in_sems.at[2])
        cp_mine.start()
        cp_r.start()

        barrier_sem = pltpu.get_barrier_semaphore()
        pl.semaphore_signal(
            barrier_sem, inc=1, device_id=partner_z,
            device_id_type=pl.DeviceIdType.MESH,
        )
        pl.semaphore_signal(
            ready_sems.at[0], inc=1, device_id=partner_x,
            device_id_type=pl.DeviceIdType.MESH,
        )
        pl.semaphore_signal(
            ready_sems.at[1], inc=1, device_id=partner_y,
            device_id_type=pl.DeviceIdType.MESH,
        )
        pl.semaphore_wait(barrier_sem, 1)

        cp_send.wait()
        ph1 = []
        for c in range(C):
            r = pltpu.make_async_remote_copy(
                src_ref=psend_ref.at[pl.ds(c * cm, cm), :],
                dst_ref=comm_ref.at[pl.ds(c * cm, cm), :],
                send_sem=send_sems.at[0, c],
                recv_sem=recv_sems.at[0, c],
                device_id=partner_z,
                device_id_type=pl.DeviceIdType.MESH,
            )
            r.start()
            ph1.append(r)

        cp_mine.wait()
        cp_r.wait()
        acc_ref[...] = pmine_ref[...] + rq_ref[...]

        ph2 = []
        out_cps = []
        for c in range(C):
            ph1[c].wait_recv()
            if c == 0:
                pl.semaphore_wait(ready_sems.at[0], 1)
                pl.semaphore_wait(ready_sems.at[1], 1)
            y = acc_ref[pl.ds(c * cm, cm), :] + comm_ref[pl.ds(c * cm, cm), :]
            inv = lax.rsqrt(jnp.mean(y * y, axis=-1, keepdims=True) + 1e-6)
            outq_ref[pl.ds(c * cm, cm), :] = y * inv * g_ref[...]

            src = outq_ref.at[pl.ds(c * cm, cm), :]
            dst = out_ref.at[pl.ds(row0 + c * cm, cm), :]
            cp = pltpu.make_async_copy(src, dst, out_sems.at[c])
            cp.start()
            out_cps.append(cp)
            for si, nbr in ((1, partner_z), (2, partner_x), (3, partner_y)):
                r = pltpu.make_async_remote_copy(
                    src_ref=src,
                    dst_ref=dst,
                    send_sem=send_sems.at[si, c],
                    recv_sem=recv_sems.at[si, c],
                    device_id=nbr,
                    device_id_type=pl.DeviceIdType.MESH,
                )
                r.start()
                ph2.append(r)

        for si, qq in ((1, 3 - q), (2, q ^ 2), (3, q ^ 1)):
            for c in range(C):
                dst = out_ref.at[pl.ds(qq * qm + c * cm, cm), :]
                rr = pltpu.make_async_remote_copy(
                    src_ref=dst,
                    dst_ref=dst,
                    send_sem=send_sems.at[si, c],
                    recv_sem=recv_sems.at[si, c],
                    device_id=partner_z,
                    device_id_type=pl.DeviceIdType.MESH,
                )
                rr.wait_recv()
        for cp in out_cps:
            cp.wait()
        for r in ph1:
            r.wait_send()
        for r in ph2:
            r.wait_send()

    return pl.pallas_call(
        body,
        out_shape=jax.ShapeDtypeStruct((m, d), jnp.float32),
        in_specs=[
            pl.BlockSpec(memory_space=pl.ANY),
            pl.BlockSpec(memory_space=pl.ANY),
            pl.BlockSpec(memory_space=pltpu.VMEM),
        ],
        out_specs=pl.BlockSpec(memory_space=pl.ANY),
        scratch_shapes=[
            pltpu.VMEM((qm, d), jnp.float32),
            pltpu.VMEM((qm, d), jnp.float32),
            pltpu.VMEM((qm, d), jnp.float32),
            pltpu.VMEM((qm, d), jnp.float32),
            pltpu.VMEM((qm, d), jnp.float32),
            pltpu.VMEM((qm, d), jnp.float32),
            pltpu.SemaphoreType.DMA((3,)),
            pltpu.SemaphoreType.DMA((C,)),
            pltpu.SemaphoreType.DMA((4, C)),
            pltpu.SemaphoreType.DMA((4, C)),
            pltpu.SemaphoreType.REGULAR((2,)),
        ],
        compiler_params=pltpu.CompilerParams(collective_id=0),
    )(partial, resid, gamma2d)


# device time: 13156 ns/iter; 1.1465x vs baseline; 1.1228x over previous
import jax
import jax.numpy as jnp
from jax import lax
from jax.experimental import pallas as pl
from jax.experimental.pallas import tpu as pltpu

C = 4


def kernel(partial, resid, gamma):
    m, d = resid.shape
    qm = m // 4
    cm = qm // C
    gamma2d = gamma.reshape(1, d)

    def body(p_ref, r_ref, g_ref, out_ref,
             comm_ref, acc_ref, psend_ref, psendb_ref, pmine_ref, rq_ref,
             outq_ref, outqb_ref, rxb_ref,
             in_sems, out_sems, send_sems, recv_sems, ready_sems):
        my_x = lax.axis_index("x")
        my_y = lax.axis_index("y")
        my_z = lax.axis_index("z")
        q = 2 * (my_x ^ my_z) + (my_y ^ my_z)
        row0 = q * qm
        partner_z = (my_x, my_y, 1 - my_z)
        partner_x = (1 - my_x, my_y, my_z)
        partner_y = (my_x, 1 - my_y, my_z)

        cp_send = pltpu.make_async_copy(
            p_ref.at[0, pl.ds((3 - q) * qm, qm), :], psend_ref, in_sems.at[0])
        cp_mine = pltpu.make_async_copy(
            p_ref.at[0, pl.ds(row0, qm), :], pmine_ref, in_sems.at[1])
        cp_r = pltpu.make_async_copy(
            r_ref.at[pl.ds(row0, qm), :], rq_ref, in_sems.at[2])
        cp_send.start()
        cp_mine.start()
        cp_r.start()

        barrier_sem = pltpu.get_barrier_semaphore()
        pl.semaphore_signal(
            barrier_sem, inc=1, device_id=partner_z,
            device_id_type=pl.DeviceIdType.MESH,
        )
        pl.semaphore_signal(
            ready_sems.at[0], inc=1, device_id=partner_x,
            device_id_type=pl.DeviceIdType.MESH,
        )
        pl.semaphore_signal(
            ready_sems.at[1], inc=1, device_id=partner_y,
            device_id_type=pl.DeviceIdType.MESH,
        )
        pl.semaphore_wait(barrier_sem, 1)

        cp_send.wait()
        psendb_ref[...] = psend_ref[...].astype(jnp.bfloat16)
        ph1 = []
        for c in range(C):
            r = pltpu.make_async_remote_copy(
                src_ref=psendb_ref.at[pl.ds(c * cm, cm), :],
                dst_ref=comm_ref.at[pl.ds(c * cm, cm), :],
                send_sem=send_sems.at[0, c],
                recv_sem=recv_sems.at[0, c],
                device_id=partner_z,
                device_id_type=pl.DeviceIdType.MESH,
            )
            r.start()
            ph1.append(r)

        cp_mine.wait()
        cp_r.wait()
        acc_ref[...] = pmine_ref[...] + rq_ref[...]

        ph2 = []
        out_cps = []
        for c in range(C):
            ph1[c].wait_recv()
            if c == 0:
                pl.semaphore_wait(ready_sems.at[0], 1)
                pl.semaphore_wait(ready_sems.at[1], 1)
            y = (acc_ref[pl.ds(c * cm, cm), :]
                 + comm_ref[pl.ds(c * cm, cm), :].astype(jnp.float32))
            inv = lax.rsqrt(jnp.mean(y * y, axis=-1, keepdims=True) + 1e-6)
            normed = y * inv * g_ref[...]
            outq_ref[pl.ds(c * cm, cm), :] = normed
            outqb_ref[pl.ds(c * cm, cm), :] = normed.astype(jnp.bfloat16)

            cp = pltpu.make_async_copy(
                outq_ref.at[pl.ds(c * cm, cm), :],
                out_ref.at[pl.ds(row0 + c * cm, cm), :],
                out_sems.at[c])
            cp.start()
            out_cps.append(cp)
            src = outqb_ref.at[pl.ds(c * cm, cm), :]
            for si, li, nbr in ((1, 0, partner_z), (2, 1, partner_x),
                                (3, 2, partner_y)):
                r = pltpu.make_async_remote_copy(
                    src_ref=src,
                    dst_ref=rxb_ref.at[li, pl.ds(c * cm, cm), :],
                    send_sem=send_sems.at[si, c],
                    recv_sem=recv_sems.at[si, c],
                    device_id=nbr,
                    device_id_type=pl.DeviceIdType.MESH,
                )
                r.start()
                ph2.append(r)

        conv_cps = []
        stages = (psend_ref, pmine_ref, rq_ref)
        for si, li, qq in ((1, 0, 3 - q), (2, 1, q ^ 2), (3, 2, q ^ 1)):
            for c in range(C):
                dst = rxb_ref.at[li, pl.ds(c * cm, cm), :]
                rr = pltpu.make_async_remote_copy(
                    src_ref=dst,
                    dst_ref=dst,
                    send_sem=send_sems.at[si, c],
                    recv_sem=recv_sems.at[si, c],
                    device_id=partner_z,
                    device_id_type=pl.DeviceIdType.MESH,
                )
                rr.wait_recv()
            stage = stages[li]
            stage[...] = rxb_ref[li].astype(jnp.float32)
            cp = pltpu.make_async_copy(
                stage, out_ref.at[pl.ds(qq * qm, qm), :], in_sems.at[li])
            cp.start()
            conv_cps.append(cp)
        for cp in conv_cps:
            cp.wait()
        for cp in out_cps:
            cp.wait()
        for r in ph1:
            r.wait_send()
        for r in ph2:
            r.wait_send()

    return pl.pallas_call(
        body,
        out_shape=jax.ShapeDtypeStruct((m, d), jnp.float32),
        in_specs=[
            pl.BlockSpec(memory_space=pl.ANY),
            pl.BlockSpec(memory_space=pl.ANY),
            pl.BlockSpec(memory_space=pltpu.VMEM),
        ],
        out_specs=pl.BlockSpec(memory_space=pl.ANY),
        scratch_shapes=[
            pltpu.VMEM((qm, d), jnp.bfloat16),
            pltpu.VMEM((qm, d), jnp.float32),
            pltpu.VMEM((qm, d), jnp.float32),
            pltpu.VMEM((qm, d), jnp.bfloat16),
            pltpu.VMEM((qm, d), jnp.float32),
            pltpu.VMEM((qm, d), jnp.float32),
            pltpu.VMEM((qm, d), jnp.float32),
            pltpu.VMEM((qm, d), jnp.bfloat16),
            pltpu.VMEM((3, qm, d), jnp.bfloat16),
            pltpu.SemaphoreType.DMA((3,)),
            pltpu.SemaphoreType.DMA((C,)),
            pltpu.SemaphoreType.DMA((4, C)),
            pltpu.SemaphoreType.DMA((4, C)),
            pltpu.SemaphoreType.REGULAR((2,)),
        ],
        compiler_params=pltpu.CompilerParams(collective_id=0),
    )(partial, resid, gamma2d)


# device time: 10507 ns/iter; 1.4355x vs baseline; 1.2521x over previous
import jax
import jax.numpy as jnp
from jax import lax
from jax.experimental import pallas as pl
from jax.experimental.pallas import tpu as pltpu

C = 4


def kernel(partial, resid, gamma):
    m, d = resid.shape
    qm = m // 4
    cm = qm // C
    gamma2d = gamma.reshape(1, d)

    def body(p_ref, r_ref, g_ref, out_ref,
             comm_ref, acc_ref, psend_ref, psendb_ref, pmine_ref, rq_ref,
             outq_ref, outqb_ref, rxb_ref, gv_ref,
             in_sems, out_sems, send_sems, recv_sems, ready_sems):
        my_x = lax.axis_index("x")
        my_y = lax.axis_index("y")
        my_z = lax.axis_index("z")
        q = 2 * (my_x ^ my_z) + (my_y ^ my_z)
        row0 = q * qm
        partner_z = (my_x, my_y, 1 - my_z)
        partner_x = (1 - my_x, my_y, my_z)
        partner_y = (my_x, 1 - my_y, my_z)

        cp_send = pltpu.make_async_copy(
            p_ref.at[0, pl.ds((3 - q) * qm, qm), :], psend_ref, in_sems.at[0])
        cp_mine = pltpu.make_async_copy(
            p_ref.at[0, pl.ds(row0, qm), :], pmine_ref, in_sems.at[1])
        cp_r = pltpu.make_async_copy(
            r_ref.at[pl.ds(row0, qm), :], rq_ref, in_sems.at[2])
        cp_g = pltpu.make_async_copy(g_ref, gv_ref, in_sems.at[3])
        cp_send.start()
        cp_mine.start()
        cp_r.start()
        cp_g.start()

        barrier_sem = pltpu.get_barrier_semaphore()
        pl.semaphore_signal(
            barrier_sem, inc=1, device_id=partner_z,
            device_id_type=pl.DeviceIdType.MESH,
        )
        pl.semaphore_signal(
            ready_sems.at[0], inc=1, device_id=partner_x,
            device_id_type=pl.DeviceIdType.MESH,
        )
        pl.semaphore_signal(
            ready_sems.at[1], inc=1, device_id=partner_y,
            device_id_type=pl.DeviceIdType.MESH,
        )
        pl.semaphore_wait(barrier_sem, 1)

        cp_send.wait()
        psendb_ref[...] = psend_ref[...].astype(jnp.bfloat16)
        ph1 = []
        for c in range(C):
            r = pltpu.make_async_remote_copy(
                src_ref=psendb_ref.at[pl.ds(c * cm, cm), :],
                dst_ref=comm_ref.at[pl.ds(c * cm, cm), :],
                send_sem=send_sems.at[0, c],
                recv_sem=recv_sems.at[0, c],
                device_id=partner_z,
                device_id_type=pl.DeviceIdType.MESH,
            )
            r.start()
            ph1.append(r)

        cp_mine.wait()
        cp_r.wait()
        cp_g.wait()
        acc_ref[...] = pmine_ref[...] + rq_ref[...]

        ph2 = []
        out_cps = []
        for c in range(C):
            ph1[c].wait_recv()
            if c == 0:
                pl.semaphore_wait(ready_sems.at[0], 1)
                pl.semaphore_wait(ready_sems.at[1], 1)
            y = (acc_ref[pl.ds(c * cm, cm), :]
                 + comm_ref[pl.ds(c * cm, cm), :].astype(jnp.float32))
            inv = lax.rsqrt(jnp.mean(y * y, axis=-1, keepdims=True) + 1e-6)
            normed = y * inv * gv_ref[...]
            outq_ref[pl.ds(c * cm, cm), :] = normed
            outqb_ref[pl.ds(c * cm, cm), :] = normed.astype(jnp.bfloat16)

            cp = pltpu.make_async_copy(
                outq_ref.at[pl.ds(c * cm, cm), :],
                out_ref.at[pl.ds(row0 + c * cm, cm), :],
                out_sems.at[c])
            cp.start()
            out_cps.append(cp)
            src = outqb_ref.at[pl.ds(c * cm, cm), :]
            for si, li, nbr in ((1, 0, partner_z), (2, 1, partner_x),
                                (3, 2, partner_y)):
                r = pltpu.make_async_remote_copy(
                    src_ref=src,
                    dst_ref=rxb_ref.at[li, pl.ds(c * cm, cm), :],
                    send_sem=send_sems.at[si, c],
                    recv_sem=recv_sems.at[si, c],
                    device_id=nbr,
                    device_id_type=pl.DeviceIdType.MESH,
                )
                r.start()
                ph2.append(r)

        conv_cps = []
        stages = (psend_ref, pmine_ref, rq_ref)
        for si, li, qq in ((1, 0, 3 - q), (2, 1, q ^ 2), (3, 2, q ^ 1)):
            for c in range(C):
                dst = rxb_ref.at[li, pl.ds(c * cm, cm), :]
                rr = pltpu.make_async_remote_copy(
                    src_ref=dst,
                    dst_ref=dst,
                    send_sem=send_sems.at[si, c],
                    recv_sem=recv_sems.at[si, c],
                    device_id=partner_z,
                    device_id_type=pl.DeviceIdType.MESH,
                )
                rr.wait_recv()
            stage = stages[li]
            stage[...] = rxb_ref[li].astype(jnp.float32)
            cp = pltpu.make_async_copy(
                stage, out_ref.at[pl.ds(qq * qm, qm), :], in_sems.at[li])
            cp.start()
            conv_cps.append(cp)
        for cp in conv_cps:
            cp.wait()
        for cp in out_cps:
            cp.wait()
        for r in ph1:
            r.wait_send()
        for r in ph2:
            r.wait_send()

    return pl.pallas_call(
        body,
        out_shape=jax.ShapeDtypeStruct((m, d), jnp.float32),
        in_specs=[
            pl.BlockSpec(memory_space=pltpu.MemorySpace.HBM),
            pl.BlockSpec(memory_space=pltpu.MemorySpace.HBM),
            pl.BlockSpec(memory_space=pltpu.MemorySpace.HBM),
        ],
        out_specs=pl.BlockSpec(memory_space=pltpu.MemorySpace.HBM),
        scratch_shapes=[
            pltpu.VMEM((qm, d), jnp.bfloat16),
            pltpu.VMEM((qm, d), jnp.float32),
            pltpu.VMEM((qm, d), jnp.float32),
            pltpu.VMEM((qm, d), jnp.bfloat16),
            pltpu.VMEM((qm, d), jnp.float32),
            pltpu.VMEM((qm, d), jnp.float32),
            pltpu.VMEM((qm, d), jnp.float32),
            pltpu.VMEM((qm, d), jnp.bfloat16),
            pltpu.VMEM((3, qm, d), jnp.bfloat16),
            pltpu.VMEM((1, d), jnp.float32),
            pltpu.SemaphoreType.DMA((4,)),
            pltpu.SemaphoreType.DMA((C,)),
            pltpu.SemaphoreType.DMA((4, C)),
            pltpu.SemaphoreType.DMA((4, C)),
            pltpu.SemaphoreType.REGULAR((2,)),
        ],
        compiler_params=pltpu.CompilerParams(collective_id=0),
    )(
        pltpu.with_memory_space_constraint(partial, pltpu.MemorySpace.HBM),
        pltpu.with_memory_space_constraint(resid, pltpu.MemorySpace.HBM),
        pltpu.with_memory_space_constraint(gamma2d, pltpu.MemorySpace.HBM),
    )


# device time: 10479 ns/iter; 1.4394x vs baseline; 1.0027x over previous
import jax
import jax.numpy as jnp
from jax import lax
from jax.experimental import pallas as pl
from jax.experimental.pallas import tpu as pltpu

C = 4


def kernel(partial, resid, gamma):
    m, d = resid.shape
    qm = m // 4
    cm = qm // C
    gamma2d = gamma.reshape(1, d)

    def body(p_ref, r_ref, g_ref, out_ref,
             comm_ref, acc_ref, psend_ref, psendb_ref, pmine_ref, rq_ref,
             outq_ref, outqb_ref, rxb_ref, gv_ref,
             in_sems, out_sems, send_sems, recv_sems, ready_sems):
        my_x = lax.axis_index("x")
        my_y = lax.axis_index("y")
        my_z = lax.axis_index("z")
        q = 2 * (my_x ^ my_z) + (my_y ^ my_z)
        row0 = q * qm
        partner_z = (my_x, my_y, 1 - my_z)
        partner_x = (1 - my_x, my_y, my_z)
        partner_y = (my_x, 1 - my_y, my_z)

        hm = qm // 2
        cp_send = [
            pltpu.make_async_copy(
                p_ref.at[0, pl.ds((3 - q) * qm + h * hm, hm), :],
                psend_ref.at[pl.ds(h * hm, hm), :],
                in_sems.at[4 + h])
            for h in range(2)
        ]
        cp_mine = pltpu.make_async_copy(
            p_ref.at[0, pl.ds(row0, qm), :], pmine_ref, in_sems.at[1])
        cp_r = pltpu.make_async_copy(
            r_ref.at[pl.ds(row0, qm), :], rq_ref, in_sems.at[2])
        cp_g = pltpu.make_async_copy(g_ref, gv_ref, in_sems.at[3])
        cp_send[0].start()
        cp_send[1].start()
        cp_mine.start()
        cp_r.start()
        cp_g.start()

        barrier_sem = pltpu.get_barrier_semaphore()
        pl.semaphore_signal(
            barrier_sem, inc=1, device_id=partner_z,
            device_id_type=pl.DeviceIdType.MESH,
        )
        pl.semaphore_signal(
            ready_sems.at[0], inc=1, device_id=partner_x,
            device_id_type=pl.DeviceIdType.MESH,
        )
        pl.semaphore_signal(
            ready_sems.at[1], inc=1, device_id=partner_y,
            device_id_type=pl.DeviceIdType.MESH,
        )
        pl.semaphore_wait(barrier_sem, 1)

        ph1 = []
        for h in range(2):
            cp_send[h].wait()
            psendb_ref[pl.ds(h * hm, hm), :] = (
                psend_ref[pl.ds(h * hm, hm), :].astype(jnp.bfloat16))
            for c in range(h * C // 2, (h + 1) * C // 2):
                r = pltpu.make_async_remote_copy(
                    src_ref=psendb_ref.at[pl.ds(c * cm, cm), :],
                    dst_ref=comm_ref.at[pl.ds(c * cm, cm), :],
                    send_sem=send_sems.at[0, c],
                    recv_sem=recv_sems.at[0, c],
                    device_id=partner_z,
                    device_id_type=pl.DeviceIdType.MESH,
                )
                r.start()
                ph1.append(r)

        cp_mine.wait()
        cp_r.wait()
        cp_g.wait()
        acc_ref[...] = pmine_ref[...] + rq_ref[...]

        ph2 = []
        out_cps = []
        for c in range(C):
            ph1[c].wait_recv()
            if c == 0:
                pl.semaphore_wait(ready_sems.at[0], 1)
                pl.semaphore_wait(ready_sems.at[1], 1)
            y = (acc_ref[pl.ds(c * cm, cm), :]
                 + comm_ref[pl.ds(c * cm, cm), :].astype(jnp.float32))
            inv = lax.rsqrt(jnp.mean(y * y, axis=-1, keepdims=True) + 1e-6)
            normed = y * inv * gv_ref[...]
            outq_ref[pl.ds(c * cm, cm), :] = normed
            outqb_ref[pl.ds(c * cm, cm), :] = normed.astype(jnp.bfloat16)

            cp = pltpu.make_async_copy(
                outq_ref.at[pl.ds(c * cm, cm), :],
                out_ref.at[pl.ds(row0 + c * cm, cm), :],
                out_sems.at[c])
            cp.start()
            out_cps.append(cp)
            src = outqb_ref.at[pl.ds(c * cm, cm), :]
            for si, li, nbr in ((1, 0, partner_z), (2, 1, partner_x),
                                (3, 2, partner_y)):
                r = pltpu.make_async_remote_copy(
                    src_ref=src,
                    dst_ref=rxb_ref.at[li, pl.ds(c * cm, cm), :],
                    send_sem=send_sems.at[si, c],
                    recv_sem=recv_sems.at[si, c],
                    device_id=nbr,
                    device_id_type=pl.DeviceIdType.MESH,
                )
                r.start()
                ph2.append(r)

        conv_cps = []
        stages = (psend_ref, pmine_ref, rq_ref)
        for si, li, qq in ((1, 0, 3 - q), (2, 1, q ^ 2), (3, 2, q ^ 1)):
            for c in range(C):
                dst = rxb_ref.at[li, pl.ds(c * cm, cm), :]
                rr = pltpu.make_async_remote_copy(
                    src_ref=dst,
                    dst_ref=dst,
                    send_sem=send_sems.at[si, c],
                    recv_sem=recv_sems.at[si, c],
                    device_id=partner_z,
                    device_id_type=pl.DeviceIdType.MESH,
                )
                rr.wait_recv()
            stage = stages[li]
            stage[...] = rxb_ref[li].astype(jnp.float32)
            cp = pltpu.make_async_copy(
                stage, out_ref.at[pl.ds(qq * qm, qm), :], in_sems.at[li])
            cp.start()
            conv_cps.append(cp)
        for cp in conv_cps:
            cp.wait()
        for cp in out_cps:
            cp.wait()
        for r in ph1:
            r.wait_send()
        for r in ph2:
            r.wait_send()

    return pl.pallas_call(
        body,
        out_shape=jax.ShapeDtypeStruct((m, d), jnp.float32),
        in_specs=[
            pl.BlockSpec(memory_space=pltpu.MemorySpace.HBM),
            pl.BlockSpec(memory_space=pltpu.MemorySpace.HBM),
            pl.BlockSpec(memory_space=pltpu.MemorySpace.HBM),
        ],
        out_specs=pl.BlockSpec(memory_space=pltpu.MemorySpace.HBM),
        scratch_shapes=[
            pltpu.VMEM((qm, d), jnp.bfloat16),
            pltpu.VMEM((qm, d), jnp.float32),
            pltpu.VMEM((qm, d), jnp.float32),
            pltpu.VMEM((qm, d), jnp.bfloat16),
            pltpu.VMEM((qm, d), jnp.float32),
            pltpu.VMEM((qm, d), jnp.float32),
            pltpu.VMEM((qm, d), jnp.float32),
            pltpu.VMEM((qm, d), jnp.bfloat16),
            pltpu.VMEM((3, qm, d), jnp.bfloat16),
            pltpu.VMEM((1, d), jnp.float32),
            pltpu.SemaphoreType.DMA((6,)),
            pltpu.SemaphoreType.DMA((C,)),
            pltpu.SemaphoreType.DMA((4, C)),
            pltpu.SemaphoreType.DMA((4, C)),
            pltpu.SemaphoreType.REGULAR((2,)),
        ],
        compiler_params=pltpu.CompilerParams(collective_id=0),
    )(
        pltpu.with_memory_space_constraint(partial, pltpu.MemorySpace.HBM),
        pltpu.with_memory_space_constraint(resid, pltpu.MemorySpace.HBM),
        pltpu.with_memory_space_constraint(gamma2d, pltpu.MemorySpace.HBM),
    )
